# baseline (speedup 1.0000x reference)
import jax
import jax.numpy as jnp
from jax.experimental import pallas as pl
from jax.experimental.pallas import tpu as pltpu

N_TOK = 8192
DIM = 4096
BM = 512


def _mm_kernel(x_ref, w_ref, b_ref, o_ref):
    xb = x_ref[...].astype(jnp.bfloat16)
    acc = jnp.dot(xb, w_ref[...], preferred_element_type=jnp.float32)
    o_ref[...] = acc + b_ref[...]


def kernel(x, weight, bias):
    wb = weight.astype(jnp.bfloat16)
    b2 = bias.reshape(1, DIM)
    return pl.pallas_call(
        _mm_kernel,
        grid=(N_TOK // BM,),
        in_specs=[
            pl.BlockSpec((BM, DIM), lambda m: (m, 0)),
            pl.BlockSpec((DIM, DIM), lambda m: (0, 0)),
            pl.BlockSpec((1, DIM), lambda m: (0, 0)),
        ],
        out_specs=pl.BlockSpec(
            (BM, DIM), lambda m: (m, 0), pipeline_mode=pl.Buffered(buffer_count=1)
        ),
        out_shape=jax.ShapeDtypeStruct((N_TOK, DIM), jnp.float32),
        compiler_params=pltpu.CompilerParams(
            allow_input_fusion=[False, True, False],
            vmem_limit_bytes=63 * 1024 * 1024,
        ),
    )(x, wb, b2)


# speedup vs baseline: 1.1536x; 1.1536x over previous
import jax
import jax.numpy as jnp
from jax.experimental import pallas as pl
from jax.experimental.pallas import tpu as pltpu

N_TOK = 8192
DIM = 4096
BM = 512
BN = 2048


def _mm_kernel(x_ref, w_ref, b_ref, o_ref):
    xb = x_ref[...].astype(jnp.bfloat16)
    acc = jnp.dot(xb, w_ref[...], preferred_element_type=jnp.float32)
    o_ref[...] = acc + b_ref[...]


def kernel(x, weight, bias):
    wb = weight.astype(jnp.bfloat16)
    b2 = bias.reshape(1, DIM)
    return pl.pallas_call(
        _mm_kernel,
        grid=(DIM // BN, N_TOK // BM),
        in_specs=[
            pl.BlockSpec((BM, DIM), lambda n, m: (m, 0)),
            pl.BlockSpec((DIM, BN), lambda n, m: (0, n)),
            pl.BlockSpec((1, BN), lambda n, m: (0, n)),
        ],
        out_specs=pl.BlockSpec((BM, BN), lambda n, m: (m, n)),
        out_shape=jax.ShapeDtypeStruct((N_TOK, DIM), jnp.float32),
        compiler_params=pltpu.CompilerParams(
            allow_input_fusion=[False, True, False],
            vmem_limit_bytes=63 * 1024 * 1024,
        ),
    )(x, wb, b2)
